# baseline (device time: 42715 ns/iter reference)
import functools
import jax
import jax.numpy as jnp
from jax import lax
from jax.experimental import pallas as pl
from jax.experimental.pallas import tpu as pltpu

B, SQ, H, D = 4, 32, 8, 128
BH = B * H
HALF = BH // 2
SCALE = D ** -0.5
NREP = 8
NROUND = 4
LOOKAHEAD = 8


def kernel(Q, K, V):
    skv = K.shape[1]
    sk = skv // NREP

    q_t = Q.transpose(0, 2, 1, 3).reshape(BH, SQ, D)

    def body(q_ref, k_ref, v_ref, out_ref,
             kh, vh, o_cur, lse_cur, o_send, o_rem, lse_rem,
             kc_sems, vc_sems,
             o_send_sems, l_send_sems, o_recv_sems, l_recv_sems):
        my_x = lax.axis_index("x")
        my_y = lax.axis_index("y")
        my_z = lax.axis_index("z")
        r = my_y * 4 + my_z
        base = r * sk

        partners = [
            (my_x, my_y, my_z ^ 2),
            (1 - my_x, my_y, my_z),
            (my_x, 1 - my_y, my_z),
            (my_x, my_y, my_z ^ 1),
        ]

        barrier = pltpu.get_barrier_semaphore()
        for p_id in partners:
            pl.semaphore_signal(barrier, inc=1, device_id=p_id,
                                device_id_type=pl.DeviceIdType.MESH)
        pl.semaphore_wait(barrier, NROUND)

        def start_dma(j):
            b, h = divmod(j, H)
            pltpu.make_async_copy(
                k_ref.at[b, pl.ds(base, sk), h, :], kh.at[j],
                kc_sems.at[j]).start()
            pltpu.make_async_copy(
                v_ref.at[b, pl.ds(base, sk), h, :], vh.at[j],
                vc_sems.at[j]).start()

        for j in range(LOOKAHEAD):
            start_dma(j)

        lse_cur[...] = jnp.zeros((SQ, 128), jnp.float32)

        def local_partial(j):
            pltpu.make_async_copy(
                k_ref.at[0, pl.ds(base, sk), 0, :], kh.at[j],
                kc_sems.at[j]).wait()
            pltpu.make_async_copy(
                v_ref.at[0, pl.ds(base, sk), 0, :], vh.at[j],
                vc_sems.at[j]).wait()
            if j + LOOKAHEAD < BH:
                start_dma(j + LOOKAHEAD)
            q = q_ref[j].astype(jnp.bfloat16)
            k = kh[j].astype(jnp.bfloat16)
            v = vh[j].astype(jnp.bfloat16)
            s = lax.dot_general(q, k, (((1,), (1,)), ((), ())),
                                preferred_element_type=jnp.float32) * SCALE
            m = jnp.max(s, axis=1, keepdims=True)
            p = jnp.exp(s - m)
            l = jnp.sum(p, axis=1, keepdims=True)
            pv = lax.dot_general(p.astype(jnp.bfloat16), v,
                                 (((1,), (0,)), ((), ())),
                                 preferred_element_type=jnp.float32)
            o_cur[pl.ds(j, 1)] = (pv / l)[None]
            lse_cur[:, j:j + 1] = m + jnp.log(l)

        def make_o_rdma(rd, lo, n, slot):
            return pltpu.make_async_remote_copy(
                src_ref=o_send.at[pl.ds(lo, n)],
                dst_ref=o_rem.at[rd, pl.ds(lo, n)],
                send_sem=o_send_sems.at[slot], recv_sem=o_recv_sems.at[slot],
                device_id=partners[rd],
                device_id_type=pl.DeviceIdType.MESH)

        def make_l_rdma(rd):
            return pltpu.make_async_remote_copy(
                src_ref=lse_cur, dst_ref=lse_rem.at[rd],
                send_sem=l_send_sems.at[rd], recv_sem=l_recv_sems.at[rd],
                device_id=partners[rd],
                device_id_type=pl.DeviceIdType.MESH)

        def combine(rd, weights, lo, n):
            w_bc = jnp.broadcast_to(
                lax.transpose(weights[:, 0:BH], (1, 0))[:, :, None],
                (BH, SQ, D))
            oc = o_cur[lo:lo + n]
            o_cur[pl.ds(lo, n)] = oc + w_bc[lo:lo + n] * (
                o_rem[rd, lo:lo + n].astype(jnp.float32) - oc)

        for j in range(HALF):
            local_partial(j)
        o_send[pl.ds(0, HALF)] = o_cur[0:HALF].astype(jnp.bfloat16)
        rdma_0a = make_o_rdma(0, 0, HALF, 0)
        rdma_0a.start()

        for j in range(HALF, BH):
            local_partial(j)
        o_send[pl.ds(HALF, HALF)] = o_cur[HALF:BH].astype(jnp.bfloat16)
        rdma_0b = make_o_rdma(0, HALF, HALF, NROUND)
        rdma_0b.start()
        rdma_l0 = make_l_rdma(0)
        rdma_l0.start()

        rdma_l0.wait()
        ls = lse_cur[...]
        lr = lse_rem[0]
        w0 = 1.0 / (1.0 + jnp.exp(ls - lr))
        mx = jnp.maximum(ls, lr)
        lse_cur[...] = mx + jnp.log(jnp.exp(ls - mx) + jnp.exp(lr - mx))
        rdma_0a.wait()
        combine(0, w0, 0, HALF)
        rdma_0b.wait()
        combine(0, w0, HALF, HALF)

        for rd in range(1, NROUND):
            o_send[...] = o_cur[...].astype(jnp.bfloat16)
            rdma_o = make_o_rdma(rd, 0, BH, rd)
            rdma_l = make_l_rdma(rd)
            rdma_o.start()
            rdma_l.start()

            rdma_l.wait()
            ls = lse_cur[...]
            lr = lse_rem[rd]
            w = 1.0 / (1.0 + jnp.exp(ls - lr))
            mx = jnp.maximum(ls, lr)
            lse_cur[...] = mx + jnp.log(jnp.exp(ls - mx) + jnp.exp(lr - mx))
            rdma_o.wait()
            combine(rd, w, 0, BH)

        out_ref[...] = o_cur[...]

        @functools.partial(pl.run_scoped,
                           second_barrier=pltpu.SemaphoreType.REGULAR)
        def _(second_barrier):
            for p_id in partners:
                pl.semaphore_signal(second_barrier, inc=1, device_id=p_id,
                                    device_id_type=pl.DeviceIdType.MESH)
            pl.semaphore_wait(second_barrier, NROUND)

    out = pl.pallas_call(
        body,
        grid=(1,),
        in_specs=[
            pl.BlockSpec((BH, SQ, D), lambda i: (0, 0, 0)),
            pl.BlockSpec(memory_space=pl.ANY),
            pl.BlockSpec(memory_space=pl.ANY),
        ],
        out_specs=pl.BlockSpec((BH, SQ, D), lambda i: (0, 0, 0)),
        out_shape=jax.ShapeDtypeStruct((BH, SQ, D), jnp.float32),
        scratch_shapes=[
            pltpu.VMEM((BH, sk, D), jnp.float32),
            pltpu.VMEM((BH, sk, D), jnp.float32),
            pltpu.VMEM((BH, SQ, D), jnp.float32),
            pltpu.VMEM((SQ, 128), jnp.float32),
            pltpu.VMEM((BH, SQ, D), jnp.bfloat16),
            pltpu.VMEM((NROUND, BH, SQ, D), jnp.bfloat16),
            pltpu.VMEM((NROUND, SQ, 128), jnp.float32),
            pltpu.SemaphoreType.DMA((BH,)),
            pltpu.SemaphoreType.DMA((BH,)),
            pltpu.SemaphoreType.DMA((NROUND + 1,)),
            pltpu.SemaphoreType.DMA((NROUND,)),
            pltpu.SemaphoreType.DMA((NROUND + 1,)),
            pltpu.SemaphoreType.DMA((NROUND,)),
        ],
        compiler_params=pltpu.CompilerParams(collective_id=0),
    )(q_t, K, V)

    return out.reshape(B, H, SQ, D).transpose(0, 2, 1, 3)


# device time: 42686 ns/iter; 1.0007x vs baseline; 1.0007x over previous
import functools
import jax
import jax.numpy as jnp
from jax import lax
from jax.experimental import pallas as pl
from jax.experimental.pallas import tpu as pltpu

B, SQ, H, D = 4, 32, 8, 128
BH = B * H
HALF = BH // 2
SCALE = D ** -0.5
NREP = 8
NROUND = 4
LOOKAHEAD = 32


def kernel(Q, K, V):
    skv = K.shape[1]
    sk = skv // NREP

    q_t = Q.transpose(0, 2, 1, 3).reshape(BH, SQ, D)

    def body(q_ref, k_ref, v_ref, out_ref,
             kh, vh, o_cur, lse_cur, o_send, o_rem, lse_rem,
             kc_sems, vc_sems,
             o_send_sems, l_send_sems, o_recv_sems, l_recv_sems):
        my_x = lax.axis_index("x")
        my_y = lax.axis_index("y")
        my_z = lax.axis_index("z")
        r = my_y * 4 + my_z
        base = r * sk

        partners = [
            (my_x, my_y, my_z ^ 2),
            (1 - my_x, my_y, my_z),
            (my_x, 1 - my_y, my_z),
            (my_x, my_y, my_z ^ 1),
        ]

        def start_dma(j):
            b, h = divmod(j, H)
            pltpu.make_async_copy(
                k_ref.at[b, pl.ds(base, sk), h, :], kh.at[j],
                kc_sems.at[j]).start()
            pltpu.make_async_copy(
                v_ref.at[b, pl.ds(base, sk), h, :], vh.at[j],
                vc_sems.at[j]).start()

        for j in range(LOOKAHEAD):
            start_dma(j)

        lse_cur[...] = jnp.zeros((SQ, 128), jnp.float32)

        def local_partial(j):
            pltpu.make_async_copy(
                k_ref.at[0, pl.ds(base, sk), 0, :], kh.at[j],
                kc_sems.at[j]).wait()
            pltpu.make_async_copy(
                v_ref.at[0, pl.ds(base, sk), 0, :], vh.at[j],
                vc_sems.at[j]).wait()
            if j + LOOKAHEAD < BH:
                start_dma(j + LOOKAHEAD)
            q = q_ref[j].astype(jnp.bfloat16)
            k = kh[j].astype(jnp.bfloat16)
            v = vh[j].astype(jnp.bfloat16)
            s = lax.dot_general(q, k, (((1,), (1,)), ((), ())),
                                preferred_element_type=jnp.float32) * SCALE
            m = jnp.max(s, axis=1, keepdims=True)
            p = jnp.exp(s - m)
            l = jnp.sum(p, axis=1, keepdims=True)
            pv = lax.dot_general(p.astype(jnp.bfloat16), v,
                                 (((1,), (0,)), ((), ())),
                                 preferred_element_type=jnp.float32)
            o_cur[pl.ds(j, 1)] = (pv / l)[None]
            lse_cur[:, j:j + 1] = m + jnp.log(l)

        def make_o_rdma(rd, lo, n, slot):
            return pltpu.make_async_remote_copy(
                src_ref=o_send.at[pl.ds(lo, n)],
                dst_ref=o_rem.at[rd, pl.ds(lo, n)],
                send_sem=o_send_sems.at[slot], recv_sem=o_recv_sems.at[slot],
                device_id=partners[rd],
                device_id_type=pl.DeviceIdType.MESH)

        def make_l_rdma(rd):
            return pltpu.make_async_remote_copy(
                src_ref=lse_cur, dst_ref=lse_rem.at[rd],
                send_sem=l_send_sems.at[rd], recv_sem=l_recv_sems.at[rd],
                device_id=partners[rd],
                device_id_type=pl.DeviceIdType.MESH)

        def combine(rd, weights, lo, n):
            w_bc = jnp.broadcast_to(
                lax.transpose(weights[:, 0:BH], (1, 0))[:, :, None],
                (BH, SQ, D))
            oc = o_cur[lo:lo + n]
            o_cur[pl.ds(lo, n)] = oc + w_bc[lo:lo + n] * (
                o_rem[rd, lo:lo + n].astype(jnp.float32) - oc)

        for j in range(HALF):
            local_partial(j)
        o_send[pl.ds(0, HALF)] = o_cur[0:HALF].astype(jnp.bfloat16)

        barrier = pltpu.get_barrier_semaphore()
        for p_id in partners:
            pl.semaphore_signal(barrier, inc=1, device_id=p_id,
                                device_id_type=pl.DeviceIdType.MESH)
        pl.semaphore_wait(barrier, NROUND)

        rdma_0a = make_o_rdma(0, 0, HALF, 0)
        rdma_0a.start()

        for j in range(HALF, BH):
            local_partial(j)
        o_send[pl.ds(HALF, HALF)] = o_cur[HALF:BH].astype(jnp.bfloat16)
        rdma_0b = make_o_rdma(0, HALF, HALF, NROUND)
        rdma_0b.start()
        rdma_l0 = make_l_rdma(0)
        rdma_l0.start()

        rdma_l0.wait()
        ls = lse_cur[...]
        lr = lse_rem[0]
        w0 = 1.0 / (1.0 + jnp.exp(ls - lr))
        mx = jnp.maximum(ls, lr)
        lse_cur[...] = mx + jnp.log(jnp.exp(ls - mx) + jnp.exp(lr - mx))
        rdma_0a.wait()
        combine(0, w0, 0, HALF)
        rdma_0b.wait()
        combine(0, w0, HALF, HALF)

        for rd in range(1, NROUND):
            o_send[...] = o_cur[...].astype(jnp.bfloat16)
            rdma_o = make_o_rdma(rd, 0, BH, rd)
            rdma_l = make_l_rdma(rd)
            rdma_o.start()
            rdma_l.start()

            rdma_l.wait()
            ls = lse_cur[...]
            lr = lse_rem[rd]
            w = 1.0 / (1.0 + jnp.exp(ls - lr))
            mx = jnp.maximum(ls, lr)
            lse_cur[...] = mx + jnp.log(jnp.exp(ls - mx) + jnp.exp(lr - mx))
            rdma_o.wait()
            combine(rd, w, 0, BH)

        out_ref[...] = o_cur[...]

        @functools.partial(pl.run_scoped,
                           second_barrier=pltpu.SemaphoreType.REGULAR)
        def _(second_barrier):
            for p_id in partners:
                pl.semaphore_signal(second_barrier, inc=1, device_id=p_id,
                                    device_id_type=pl.DeviceIdType.MESH)
            pl.semaphore_wait(second_barrier, NROUND)

    out = pl.pallas_call(
        body,
        grid=(1,),
        in_specs=[
            pl.BlockSpec((BH, SQ, D), lambda i: (0, 0, 0)),
            pl.BlockSpec(memory_space=pl.ANY),
            pl.BlockSpec(memory_space=pl.ANY),
        ],
        out_specs=pl.BlockSpec((BH, SQ, D), lambda i: (0, 0, 0)),
        out_shape=jax.ShapeDtypeStruct((BH, SQ, D), jnp.float32),
        scratch_shapes=[
            pltpu.VMEM((BH, sk, D), jnp.float32),
            pltpu.VMEM((BH, sk, D), jnp.float32),
            pltpu.VMEM((BH, SQ, D), jnp.float32),
            pltpu.VMEM((SQ, 128), jnp.float32),
            pltpu.VMEM((BH, SQ, D), jnp.bfloat16),
            pltpu.VMEM((NROUND, BH, SQ, D), jnp.bfloat16),
            pltpu.VMEM((NROUND, SQ, 128), jnp.float32),
            pltpu.SemaphoreType.DMA((BH,)),
            pltpu.SemaphoreType.DMA((BH,)),
            pltpu.SemaphoreType.DMA((NROUND + 1,)),
            pltpu.SemaphoreType.DMA((NROUND,)),
            pltpu.SemaphoreType.DMA((NROUND + 1,)),
            pltpu.SemaphoreType.DMA((NROUND,)),
        ],
        compiler_params=pltpu.CompilerParams(collective_id=0),
    )(q_t, K, V)

    return out.reshape(B, H, SQ, D).transpose(0, 2, 1, 3)
